# baseline (speedup 1.0000x reference)
import functools

import jax
import jax.numpy as jnp
from jax import lax
from jax.experimental import pallas as pl
from jax.experimental.pallas import tpu as pltpu
from jax.experimental.pallas import tpu_sc as plsc

_B = 16384
_D = 128
_LANES = 16
_NC = 2
_NS = 16
_NW = _NC * _NS
_ROWS_PER_W = _B // _NW
_CHUNK = 64
_NCHUNK = _ROWS_PER_W // _CHUNK


def _sc_gather_mult(in_repr, idx2d, lambdas):
    mesh = plsc.VectorSubcoreMesh(core_axis_name="c", subcore_axis_name="s")

    @functools.partial(
        pl.kernel,
        out_type=jax.ShapeDtypeStruct((_B, _D), jnp.float32),
        mesh=mesh,
        scratch_types=[
            pltpu.VMEM((_NCHUNK, _CHUNK), jnp.int32),
            pltpu.VMEM((_NCHUNK, _CHUNK, _D), jnp.float32),
            pltpu.VMEM((2, _CHUNK, _D), jnp.float32),
            pltpu.SemaphoreType.DMA,
            pltpu.SemaphoreType.DMA,
            pltpu.SemaphoreType.DMA,
        ],
    )
    def k(in_hbm, idx_hbm, lam_hbm, out_hbm, idx_v, lam3, x2, gsem, xsem, osem):
        wid = lax.axis_index("s") * _NC + lax.axis_index("c")
        base = wid * _ROWS_PER_W

        pltpu.async_copy(in_hbm.at[pl.ds(base, _CHUNK)], x2.at[0], xsem)
        pltpu.sync_copy(idx_hbm.at[pl.ds(wid * _NCHUNK, _NCHUNK)], idx_v)
        for c in range(_NCHUNK):
            pltpu.async_copy(lam_hbm.at[idx_v.at[c]], lam3.at[c], gsem)

        @pl.loop(0, _NCHUNK)
        def _(c):
            b = lax.rem(c, 2)

            @pl.when(c + 1 < _NCHUNK)
            def _():
                pltpu.async_copy(
                    in_hbm.at[pl.ds(base + (c + 1) * _CHUNK, _CHUNK)],
                    x2.at[1 - b],
                    xsem,
                )

            pltpu.make_async_copy(lam_hbm.at[idx_v.at[c]], lam3.at[c], gsem).wait()
            pltpu.make_async_copy(
                in_hbm.at[pl.ds(base + c * _CHUNK, _CHUNK)], x2.at[b], xsem
            ).wait()

            @pl.loop(0, _CHUNK)
            def _(r):
                for c0 in range(0, _D, _LANES):
                    lam3[c, r, pl.ds(c0, _LANES)] = (
                        lam3[c, r, pl.ds(c0, _LANES)] * x2[b, r, pl.ds(c0, _LANES)]
                    )

            pltpu.async_copy(
                lam3.at[c], out_hbm.at[pl.ds(base + c * _CHUNK, _CHUNK)], osem
            )

        @pl.loop(0, _NCHUNK)
        def _(c):
            pltpu.make_async_copy(
                lam3.at[c], out_hbm.at[pl.ds(base + c * _CHUNK, _CHUNK)], osem
            ).wait()

    return k(in_repr, idx2d, lambdas)


def kernel(in_repr, group_idx, lambdas):
    idx2d = group_idx.astype(jnp.int32).reshape(_B // _CHUNK, _CHUNK)
    return _sc_gather_mult(in_repr, idx2d, lambdas)


# speedup vs baseline: 1.4591x; 1.4591x over previous
import functools

import jax
import jax.numpy as jnp
from jax import lax
from jax.experimental import pallas as pl
from jax.experimental.pallas import tpu as pltpu
from jax.experimental.pallas import tpu_sc as plsc

_B = 16384
_D = 128
_LANES = 16
_NC = 2
_NS = 16
_NW = _NC * _NS
_ROWS_PER_W = _B // _NW
_CHUNK = 64
_NCHUNK = _ROWS_PER_W // _CHUNK


def _sc_gather_mult(in_repr, idx2d, lambdas):
    mesh = plsc.VectorSubcoreMesh(core_axis_name="c", subcore_axis_name="s")

    lam_scratch = [pltpu.VMEM((_CHUNK, _D), jnp.float32) for _ in range(_NCHUNK)]
    x_scratch = [pltpu.VMEM((_CHUNK, _D), jnp.float32) for _ in range(2)]
    sems = [pltpu.SemaphoreType.DMA for _ in range(2 * _NCHUNK + 2)]

    @functools.partial(
        pl.kernel,
        out_type=jax.ShapeDtypeStruct((_B, _D), jnp.float32),
        mesh=mesh,
        scratch_types=(
            [pltpu.VMEM((_NCHUNK, _CHUNK), jnp.int32)]
            + lam_scratch + x_scratch + sems
        ),
    )
    def k(in_hbm, idx_hbm, lam_hbm, out_hbm, idx_v, *bufs):
        lam = list(bufs[:_NCHUNK])
        xb = list(bufs[_NCHUNK:_NCHUNK + 2])
        gsem = list(bufs[_NCHUNK + 2:2 * _NCHUNK + 2])
        xsem = list(bufs[2 * _NCHUNK + 2:2 * _NCHUNK + 4])
        osem = list(bufs[2 * _NCHUNK + 4:])

        wid = lax.axis_index("s") * _NC + lax.axis_index("c")
        base = wid * _ROWS_PER_W

        xgets = [None] * _NCHUNK
        puts = [None] * _NCHUNK

        def start_x(c):
            xgets[c] = pltpu.async_copy(
                in_hbm.at[pl.ds(base + c * _CHUNK, _CHUNK)], xb[c % 2], xsem[c % 2]
            )

        start_x(0)
        pltpu.sync_copy(idx_hbm.at[pl.ds(wid * _NCHUNK, _NCHUNK)], idx_v)

        gets = [
            pltpu.async_copy(lam_hbm.at[idx_v.at[c]], lam[c], gsem[c])
            for c in range(_NCHUNK)
        ]
        for c in range(_NCHUNK):
            xv = xb[c % 2]
            if c + 1 < _NCHUNK:
                start_x(c + 1)
            gets[c].wait()
            xgets[c].wait()

            @pl.loop(0, _CHUNK)
            def _(r):
                for c0 in range(0, _D, _LANES):
                    lam[c][r, pl.ds(c0, _LANES)] = (
                        lam[c][r, pl.ds(c0, _LANES)] * xv[r, pl.ds(c0, _LANES)]
                    )

            puts[c] = pltpu.async_copy(
                lam[c], out_hbm.at[pl.ds(base + c * _CHUNK, _CHUNK)], osem[c % 2]
            )
        for c in range(_NCHUNK):
            puts[c].wait()

    return k(in_repr, idx2d, lambdas)


def kernel(in_repr, group_idx, lambdas):
    idx2d = group_idx.astype(jnp.int32).reshape(_B // _CHUNK, _CHUNK)
    return _sc_gather_mult(in_repr, idx2d, lambdas)


# speedup vs baseline: 1.4871x; 1.0192x over previous
import functools

import jax
import jax.numpy as jnp
from jax import lax
from jax.experimental import pallas as pl
from jax.experimental.pallas import tpu as pltpu
from jax.experimental.pallas import tpu_sc as plsc

_B = 16384
_D = 128
_LANES = 16
_NC = 2
_NS = 16
_NW = _NC * _NS
_ROWS_PER_W = _B // _NW
_CHUNK = 128
_NCHUNK = _ROWS_PER_W // _CHUNK


def _sc_gather_mult(in_repr, idx2d, lambdas):
    mesh = plsc.VectorSubcoreMesh(core_axis_name="c", subcore_axis_name="s")

    lam_scratch = [pltpu.VMEM((_CHUNK, _D), jnp.float32) for _ in range(_NCHUNK)]
    x_scratch = [pltpu.VMEM((_CHUNK, _D), jnp.float32) for _ in range(2)]
    sems = [pltpu.SemaphoreType.DMA for _ in range(2 * _NCHUNK + 2)]

    @functools.partial(
        pl.kernel,
        out_type=jax.ShapeDtypeStruct((_B, _D), jnp.float32),
        mesh=mesh,
        scratch_types=(
            [pltpu.VMEM((_NCHUNK, _CHUNK), jnp.int32)]
            + lam_scratch + x_scratch + sems
        ),
    )
    def k(in_hbm, idx_hbm, lam_hbm, out_hbm, idx_v, *bufs):
        lam = list(bufs[:_NCHUNK])
        xb = list(bufs[_NCHUNK:_NCHUNK + 2])
        gsem = list(bufs[_NCHUNK + 2:2 * _NCHUNK + 2])
        xsem = list(bufs[2 * _NCHUNK + 2:2 * _NCHUNK + 4])
        osem = list(bufs[2 * _NCHUNK + 4:])

        wid = lax.axis_index("s") * _NC + lax.axis_index("c")
        base = wid * _ROWS_PER_W

        xgets = [None] * _NCHUNK
        puts = [None] * _NCHUNK

        def start_x(c):
            xgets[c] = pltpu.async_copy(
                in_hbm.at[pl.ds(base + c * _CHUNK, _CHUNK)], xb[c % 2], xsem[c % 2]
            )

        start_x(0)
        start_x(1)
        pltpu.sync_copy(idx_hbm.at[pl.ds(wid * _NCHUNK, _NCHUNK)], idx_v)

        gets = [
            pltpu.async_copy(lam_hbm.at[idx_v.at[c]], lam[c], gsem[c])
            for c in range(_NCHUNK)
        ]
        for c in range(_NCHUNK):
            xv = xb[c % 2]
            gets[c].wait()
            xgets[c].wait()

            @pl.loop(0, _CHUNK)
            def _(r):
                for c0 in range(0, _D, _LANES):
                    lam[c][r, pl.ds(c0, _LANES)] = (
                        lam[c][r, pl.ds(c0, _LANES)] * xv[r, pl.ds(c0, _LANES)]
                    )

            puts[c] = pltpu.async_copy(
                lam[c], out_hbm.at[pl.ds(base + c * _CHUNK, _CHUNK)], osem[c % 2]
            )
            if c + 2 < _NCHUNK:
                start_x(c + 2)
        for c in range(_NCHUNK):
            puts[c].wait()

    return k(in_repr, idx2d, lambdas)


def kernel(in_repr, group_idx, lambdas):
    idx2d = group_idx.astype(jnp.int32).reshape(_B // _CHUNK, _CHUNK)
    return _sc_gather_mult(in_repr, idx2d, lambdas)


# speedup vs baseline: 1.5464x; 1.0399x over previous
import functools

import jax
import jax.numpy as jnp
from jax import lax
from jax.experimental import pallas as pl
from jax.experimental.pallas import tpu as pltpu
from jax.experimental.pallas import tpu_sc as plsc

_B = 16384
_D = 128
_LANES = 16
_NC = 2
_NS = 16
_NW = _NC * _NS
_ROWS_PER_W = _B // _NW
_SIZES = (64, 128, 128, 128, 64)
_OFFS = tuple(sum(_SIZES[:i]) for i in range(len(_SIZES)))
_NCHUNK = len(_SIZES)
assert sum(_SIZES) == _ROWS_PER_W


def _sc_gather_mult(in_repr, group_idx, lambdas):
    mesh = plsc.VectorSubcoreMesh(core_axis_name="c", subcore_axis_name="s")

    lam_scratch = [pltpu.VMEM((s, _D), jnp.float32) for s in _SIZES]
    x_scratch = [pltpu.VMEM((max(_SIZES), _D), jnp.float32) for _ in range(2)]
    sems = [pltpu.SemaphoreType.DMA for _ in range(2 * _NCHUNK + 2)]

    @functools.partial(
        pl.kernel,
        out_type=jax.ShapeDtypeStruct((_B, _D), jnp.float32),
        mesh=mesh,
        scratch_types=(
            [pltpu.VMEM((_ROWS_PER_W,), jnp.int32)]
            + lam_scratch + x_scratch + sems
        ),
    )
    def k(in_hbm, idx_hbm, lam_hbm, out_hbm, idx_v, *bufs):
        lam = list(bufs[:_NCHUNK])
        xb = list(bufs[_NCHUNK:_NCHUNK + 2])
        gsem = list(bufs[_NCHUNK + 2:2 * _NCHUNK + 2])
        xsem = list(bufs[2 * _NCHUNK + 2:2 * _NCHUNK + 4])
        osem = list(bufs[2 * _NCHUNK + 4:])

        wid = lax.axis_index("s") * _NC + lax.axis_index("c")
        base = wid * _ROWS_PER_W

        xgets = [None] * _NCHUNK
        puts = [None] * _NCHUNK

        def start_x(c):
            xgets[c] = pltpu.async_copy(
                in_hbm.at[pl.ds(base + _OFFS[c], _SIZES[c])],
                xb[c % 2].at[pl.ds(0, _SIZES[c])],
                xsem[c % 2],
            )

        start_x(0)
        start_x(1)
        pltpu.sync_copy(idx_hbm.at[pl.ds(base, _ROWS_PER_W)], idx_v)

        gets = [
            pltpu.async_copy(
                lam_hbm.at[idx_v.at[pl.ds(_OFFS[c], _SIZES[c])]], lam[c], gsem[c]
            )
            for c in range(_NCHUNK)
        ]
        for c in range(_NCHUNK):
            xv = xb[c % 2]
            gets[c].wait()
            xgets[c].wait()

            @pl.loop(0, _SIZES[c])
            def _(r):
                for c0 in range(0, _D, _LANES):
                    lam[c][r, pl.ds(c0, _LANES)] = (
                        lam[c][r, pl.ds(c0, _LANES)] * xv[r, pl.ds(c0, _LANES)]
                    )

            puts[c] = pltpu.async_copy(
                lam[c], out_hbm.at[pl.ds(base + _OFFS[c], _SIZES[c])], osem[c % 2]
            )
            if c + 2 < _NCHUNK:
                start_x(c + 2)
        for c in range(_NCHUNK):
            puts[c].wait()

    return k(in_repr, group_idx, lambdas)


def kernel(in_repr, group_idx, lambdas):
    return _sc_gather_mult(in_repr, group_idx.astype(jnp.int32), lambdas)
